# initial kernel scaffold (unmeasured)
import jax
import jax.numpy as jnp
from jax import lax
from jax.experimental import pallas as pl
from jax.experimental.pallas import tpu as pltpu

B, S, D = 2, 512, 2048
H, Dh, Dr = 16, 128, 32
DC_HALF = 128
SCALE = (Dh + Dr) ** -0.5


def _dot(a, b):
    return jnp.dot(a, b, preferred_element_type=jnp.float32)


def _dot_t(a, b):
    return lax.dot_general(
        a, b, (((1,), (1,)), ((), ())), preferred_element_type=jnp.float32
    )


def _kv_body(x_ref, wdkv_ref, wuk_ref, wuv_ref, k_ref, v_ref,
             c_loc, c_rem, wuk_rem, wuv_rem, send_sems, recv_sems):
    my_x = lax.axis_index("x")
    my_y = lax.axis_index("y")
    nbr = (my_x, 1 - my_y)

    barrier = pltpu.get_barrier_semaphore()
    pl.semaphore_signal(
        barrier, inc=1, device_id=nbr, device_id_type=pl.DeviceIdType.MESH
    )
    pl.semaphore_wait(barrier, 1)

    xm = x_ref[...].reshape(B * S, D)
    c_loc[...] = _dot(xm, wdkv_ref[...])

    rdmas = []
    for src, dst, i in (
        (c_loc, c_rem, 0),
        (wuk_ref, wuk_rem, 1),
        (wuv_ref, wuv_rem, 2),
    ):
        rdma = pltpu.make_async_remote_copy(
            src_ref=src,
            dst_ref=dst,
            send_sem=send_sems.at[i],
            recv_sem=recv_sems.at[i],
            device_id=nbr,
            device_id_type=pl.DeviceIdType.MESH,
        )
        rdma.start()
        rdmas.append(rdma)

    k_part = _dot(c_loc[...], wuk_ref[...])
    v_part = _dot(c_loc[...], wuv_ref[...])

    for rdma in rdmas:
        rdma.wait()

    k = k_part + _dot(c_rem[...], wuk_rem[...])
    v = v_part + _dot(c_rem[...], wuv_rem[...])
    k_ref[...] = k.reshape(B, S, D)
    v_ref[...] = v.reshape(B, S, D)


def _compute_kv(x, Wdkv, Wuk, Wuv):
    return pl.pallas_call(
        _kv_body,
        out_shape=(
            jax.ShapeDtypeStruct((B, S, D), jnp.float32),
            jax.ShapeDtypeStruct((B, S, D), jnp.float32),
        ),
        in_specs=[pl.BlockSpec(memory_space=pltpu.VMEM)] * 4,
        out_specs=(
            pl.BlockSpec(memory_space=pltpu.VMEM),
            pl.BlockSpec(memory_space=pltpu.VMEM),
        ),
        scratch_shapes=[
            pltpu.VMEM((B * S, DC_HALF), jnp.float32),
            pltpu.VMEM((B * S, DC_HALF), jnp.float32),
            pltpu.VMEM((DC_HALF, D), jnp.float32),
            pltpu.VMEM((DC_HALF, D), jnp.float32),
            pltpu.SemaphoreType.DMA((3,)),
            pltpu.SemaphoreType.DMA((3,)),
        ],
        compiler_params=pltpu.CompilerParams(
            collective_id=0, has_side_effects=True
        ),
    )(x, Wdkv, Wuk, Wuv)


def _attn_body(x_ref, k_ref, v_ref, wq_ref, wqr_ref, wkr_ref, wo_ref,
               out_ref, kr_scratch):
    h = pl.program_id(1)
    xb = x_ref[0]

    @pl.when(h == 0)
    def _():
        kr_scratch[...] = _dot(xb, wkr_ref[...])

    q = _dot(xb, wq_ref[...])
    qr = _dot(xb, wqr_ref[...])

    scores = (_dot_t(q, k_ref[0]) + _dot_t(qr, kr_scratch[...])) * SCALE
    m = jnp.max(scores, axis=-1, keepdims=True)
    p = jnp.exp(scores - m)
    p = p / jnp.sum(p, axis=-1, keepdims=True)
    o = _dot(p, v_ref[0])
    contrib = _dot(o, wo_ref[...])

    @pl.when(h == 0)
    def _():
        out_ref[0] = contrib

    @pl.when(h > 0)
    def _():
        out_ref[0] += contrib


def _attention(x, K, V, Wq, Wqr, Wkr, Wo):
    return pl.pallas_call(
        _attn_body,
        grid=(B, H),
        out_shape=jax.ShapeDtypeStruct((B, S, D), jnp.float32),
        in_specs=[
            pl.BlockSpec((1, S, D), lambda b, h: (b, 0, 0)),
            pl.BlockSpec((1, S, Dh), lambda b, h: (b, 0, h)),
            pl.BlockSpec((1, S, Dh), lambda b, h: (b, 0, h)),
            pl.BlockSpec((D, Dh), lambda b, h: (0, h)),
            pl.BlockSpec((D, Dr), lambda b, h: (0, h)),
            pl.BlockSpec((D, Dr), lambda b, h: (0, 0)),
            pl.BlockSpec((Dh, D), lambda b, h: (h, 0)),
        ],
        out_specs=pl.BlockSpec((1, S, D), lambda b, h: (b, 0, 0)),
        scratch_shapes=[pltpu.VMEM((S, Dr), jnp.float32)],
        compiler_params=pltpu.CompilerParams(
            dimension_semantics=("arbitrary", "arbitrary"),
        ),
    )(x, K, V, Wq, Wqr, Wkr, Wo)


def kernel(x, Wdkv, Wuk, Wuv, Wq, Wqr, Wkr, Wo):
    K, V = _compute_kv(x, Wdkv, Wuk, Wuv)
    return _attention(x, K, V, Wq, Wqr, Wkr, Wo)


# baseline (device time: 170404 ns/iter reference)
import jax
import jax.numpy as jnp
from jax import lax
from jax.experimental import pallas as pl
from jax.experimental.pallas import tpu as pltpu

B, S, D = 2, 512, 2048
H, Dh, Dr = 16, 128, 32
DC_HALF = 128
SCALE = (Dh + Dr) ** -0.5


def _dot(a, b):
    return jnp.dot(a, b, preferred_element_type=jnp.float32)


def _dot_t(a, b):
    return lax.dot_general(
        a, b, (((1,), (1,)), ((), ())), preferred_element_type=jnp.float32
    )


def _kv_body(x_ref, wdkv_ref, wuk_ref, wuv_ref, k_ref, v_ref,
             c_loc, c_rem, wuk_rem, wuv_rem, send_sems, recv_sems):
    my_x = lax.axis_index("x")
    my_y = lax.axis_index("y")
    nbr = (my_x, 1 - my_y)

    barrier = pltpu.get_barrier_semaphore()
    pl.semaphore_signal(
        barrier, inc=1, device_id=nbr, device_id_type=pl.DeviceIdType.MESH
    )
    pl.semaphore_wait(barrier, 1)

    xm = x_ref[...].reshape(B * S, D)
    c_loc[...] = _dot(xm, wdkv_ref[...])

    rdmas = []
    for src, dst, i in (
        (c_loc, c_rem, 0),
        (wuk_ref, wuk_rem, 1),
        (wuv_ref, wuv_rem, 2),
    ):
        rdma = pltpu.make_async_remote_copy(
            src_ref=src,
            dst_ref=dst,
            send_sem=send_sems.at[i],
            recv_sem=recv_sems.at[i],
            device_id=nbr,
            device_id_type=pl.DeviceIdType.MESH,
        )
        rdma.start()
        rdmas.append(rdma)

    k_part = _dot(c_loc[...], wuk_ref[...])
    v_part = _dot(c_loc[...], wuv_ref[...])

    for rdma in rdmas:
        rdma.wait()

    k = k_part + _dot(c_rem[...], wuk_rem[...])
    v = v_part + _dot(c_rem[...], wuv_rem[...])
    k_ref[...] = k.reshape(B, S, D)
    v_ref[...] = v.reshape(B, S, D)


def _compute_kv(x, Wdkv, Wuk, Wuv):
    return pl.pallas_call(
        _kv_body,
        out_shape=(
            jax.ShapeDtypeStruct((B, S, D), jnp.float32),
            jax.ShapeDtypeStruct((B, S, D), jnp.float32),
        ),
        in_specs=[pl.BlockSpec(memory_space=pltpu.VMEM)] * 4,
        out_specs=(
            pl.BlockSpec(memory_space=pltpu.VMEM),
            pl.BlockSpec(memory_space=pltpu.VMEM),
        ),
        scratch_shapes=[
            pltpu.VMEM((B * S, DC_HALF), jnp.float32),
            pltpu.VMEM((B * S, DC_HALF), jnp.float32),
            pltpu.VMEM((DC_HALF, D), jnp.float32),
            pltpu.VMEM((DC_HALF, D), jnp.float32),
            pltpu.SemaphoreType.DMA((3,)),
            pltpu.SemaphoreType.DMA((3,)),
        ],
        compiler_params=pltpu.CompilerParams(
            collective_id=0, has_side_effects=True
        ),
    )(x, Wdkv, Wuk, Wuv)


def _attn_body(x_ref, k_ref, v_ref, wq_ref, wqrt_ref, wkr_ref, wo_ref,
               out_ref, kr_scratch):
    h = pl.program_id(1)
    xb = x_ref[0]

    @pl.when(h == 0)
    def _():
        kr_scratch[...] = _dot(xb, wkr_ref[...])

    q = _dot(xb, wq_ref[...])
    qr = _dot_t(xb, wqrt_ref[0])

    scores = (_dot_t(q, k_ref[0]) + _dot_t(qr, kr_scratch[...])) * SCALE
    m = jnp.max(scores, axis=-1, keepdims=True)
    p = jnp.exp(scores - m)
    p = p / jnp.sum(p, axis=-1, keepdims=True)
    o = _dot(p, v_ref[0])
    contrib = _dot(o, wo_ref[...])

    @pl.when(h == 0)
    def _():
        out_ref[0] = contrib

    @pl.when(h > 0)
    def _():
        out_ref[0] += contrib


def _attention(x, K, V, Wq, Wqr, Wkr, Wo):
    Wqr_t = Wqr.T.reshape(H, Dr, D)
    return pl.pallas_call(
        _attn_body,
        grid=(B, H),
        out_shape=jax.ShapeDtypeStruct((B, S, D), jnp.float32),
        in_specs=[
            pl.BlockSpec((1, S, D), lambda b, h: (b, 0, 0)),
            pl.BlockSpec((1, S, Dh), lambda b, h: (b, 0, h)),
            pl.BlockSpec((1, S, Dh), lambda b, h: (b, 0, h)),
            pl.BlockSpec((D, Dh), lambda b, h: (0, h)),
            pl.BlockSpec((1, Dr, D), lambda b, h: (h, 0, 0)),
            pl.BlockSpec((D, Dr), lambda b, h: (0, 0)),
            pl.BlockSpec((Dh, D), lambda b, h: (h, 0)),
        ],
        out_specs=pl.BlockSpec((1, S, D), lambda b, h: (b, 0, 0)),
        scratch_shapes=[pltpu.VMEM((S, Dr), jnp.float32)],
        compiler_params=pltpu.CompilerParams(
            dimension_semantics=("arbitrary", "arbitrary"),
        ),
    )(x, K, V, Wq, Wqr_t, Wkr, Wo)


def kernel(x, Wdkv, Wuk, Wuv, Wq, Wqr, Wkr, Wo):
    K, V = _compute_kv(x, Wdkv, Wuk, Wuv)
    return _attention(x, K, V, Wq, Wqr, Wkr, Wo)


# device time: 141216 ns/iter; 1.2067x vs baseline; 1.2067x over previous
import jax
import jax.numpy as jnp
from jax import lax
from jax.experimental import pallas as pl
from jax.experimental.pallas import tpu as pltpu

B, S, D = 2, 512, 2048
H, Dh, Dr = 16, 128, 32
NJ = H // 4
G4 = NJ * Dh
DC_HALF = 128
SCALE = (Dh + Dr) ** -0.5


def _dot(a, b):
    return jnp.dot(a, b, preferred_element_type=jnp.float32)


def _dot_t(a, b):
    return lax.dot_general(
        a, b, (((1,), (1,)), ((), ())), preferred_element_type=jnp.float32
    )


def _kv_body(x_ref, wdkv_ref, wuk_ref, wuv_ref, k_ref, v_ref,
             c_loc, c_rem, wuk_rem, wuv_rem, send_sems, recv_sems):
    my_x = lax.axis_index("x")
    my_y = lax.axis_index("y")
    nbr = (my_x, 1 - my_y)

    barrier = pltpu.get_barrier_semaphore()
    pl.semaphore_signal(
        barrier, inc=1, device_id=nbr, device_id_type=pl.DeviceIdType.MESH
    )
    pl.semaphore_wait(barrier, 1)

    xm = x_ref[...].reshape(B * S, D)
    c_loc[...] = _dot(xm, wdkv_ref[...])

    rdmas = []
    for src, dst, i in (
        (c_loc, c_rem, 0),
        (wuk_ref, wuk_rem, 1),
        (wuv_ref, wuv_rem, 2),
    ):
        rdma = pltpu.make_async_remote_copy(
            src_ref=src,
            dst_ref=dst,
            send_sem=send_sems.at[i],
            recv_sem=recv_sems.at[i],
            device_id=nbr,
            device_id_type=pl.DeviceIdType.MESH,
        )
        rdma.start()
        rdmas.append(rdma)

    k_part = _dot(c_loc[...], wuk_ref[...])
    v_part = _dot(c_loc[...], wuv_ref[...])

    for rdma in rdmas:
        rdma.wait()

    k = k_part + _dot(c_rem[...], wuk_rem[...])
    v = v_part + _dot(c_rem[...], wuv_rem[...])
    k_ref[...] = k.reshape(B, S, D)
    v_ref[...] = v.reshape(B, S, D)


def _compute_kv(x, Wdkv, Wuk, Wuv):
    return pl.pallas_call(
        _kv_body,
        out_shape=(
            jax.ShapeDtypeStruct((B, S, D), jnp.float32),
            jax.ShapeDtypeStruct((B, S, D), jnp.float32),
        ),
        in_specs=[pl.BlockSpec(memory_space=pltpu.VMEM)] * 4,
        out_specs=(
            pl.BlockSpec(memory_space=pltpu.VMEM),
            pl.BlockSpec(memory_space=pltpu.VMEM),
        ),
        scratch_shapes=[
            pltpu.VMEM((B * S, DC_HALF), jnp.float32),
            pltpu.VMEM((B * S, DC_HALF), jnp.float32),
            pltpu.VMEM((DC_HALF, D), jnp.float32),
            pltpu.VMEM((DC_HALF, D), jnp.float32),
            pltpu.SemaphoreType.DMA((3,)),
            pltpu.SemaphoreType.DMA((3,)),
        ],
        compiler_params=pltpu.CompilerParams(
            collective_id=0, has_side_effects=True
        ),
    )(x, Wdkv, Wuk, Wuv)


def _attn_body(g_ref, x_ref, k_ref, v_ref, wq_ref, wqrt_ref, wkr_ref,
               o_ref, kr_scratch):
    del g_ref
    j = pl.program_id(1)
    xb = x_ref[0]

    @pl.when(j == 0)
    def _():
        kr_scratch[...] = _dot(xb, wkr_ref[...])

    q = _dot(xb, wq_ref[...])
    qr = _dot_t(xb, wqrt_ref[0])

    scores = (_dot_t(q, k_ref[0]) + _dot_t(qr, kr_scratch[...])) * SCALE
    m = jnp.max(scores, axis=-1, keepdims=True)
    p = jnp.exp(scores - m)
    p = p / jnp.sum(p, axis=-1, keepdims=True)
    o_ref[0] = _dot(p, v_ref[0])


def _attention(g, x, K, V, Wq, Wqr, Wkr):
    Wqr_t = Wqr.T.reshape(H, Dr, D)
    grid_spec = pltpu.PrefetchScalarGridSpec(
        num_scalar_prefetch=1,
        grid=(B, NJ),
        in_specs=[
            pl.BlockSpec((1, S, D), lambda b, j, g: (b, 0, 0)),
            pl.BlockSpec((1, S, Dh), lambda b, j, g: (b, 0, g[0] * NJ + j)),
            pl.BlockSpec((1, S, Dh), lambda b, j, g: (b, 0, g[0] * NJ + j)),
            pl.BlockSpec((D, Dh), lambda b, j, g: (0, g[0] * NJ + j)),
            pl.BlockSpec((1, Dr, D), lambda b, j, g: (g[0] * NJ + j, 0, 0)),
            pl.BlockSpec((D, Dr), lambda b, j, g: (0, 0)),
        ],
        out_specs=pl.BlockSpec((1, S, Dh), lambda b, j, g: (b, 0, j)),
        scratch_shapes=[pltpu.VMEM((S, Dr), jnp.float32)],
    )
    return pl.pallas_call(
        _attn_body,
        grid_spec=grid_spec,
        out_shape=jax.ShapeDtypeStruct((B, S, G4), jnp.float32),
        compiler_params=pltpu.CompilerParams(
            dimension_semantics=("arbitrary", "arbitrary"),
        ),
    )(g, x, K, V, Wq, Wqr_t, Wkr)


def _gather_body(o_ref, wo_ref, out_ref, oy, ox0, ox1, send_sems, recv_sems):
    my_x = lax.axis_index("x")
    my_y = lax.axis_index("y")
    g = 2 * my_x + my_y
    y_nbr = (my_x, 1 - my_y)
    x_nbr = (1 - my_x, my_y)

    barrier = pltpu.get_barrier_semaphore()
    for nbr in (y_nbr, x_nbr):
        pl.semaphore_signal(
            barrier, inc=1, device_id=nbr, device_id_type=pl.DeviceIdType.MESH
        )
    pl.semaphore_wait(barrier, 2)

    def remote_copy(src, dst, i, dev):
        return pltpu.make_async_remote_copy(
            src_ref=src, dst_ref=dst,
            send_sem=send_sems.at[i], recv_sem=recv_sems.at[i],
            device_id=dev, device_id_type=pl.DeviceIdType.MESH,
        )

    r_y = remote_copy(o_ref, oy, 0, y_nbr)
    r_x0 = remote_copy(o_ref, ox0, 1, x_nbr)
    r_y.start()
    r_x0.start()

    def wo_rows(grp):
        return wo_ref[pl.ds(grp * G4, G4), :]

    out_ref[...] = _dot(o_ref[...].reshape(B * S, G4), wo_rows(g)).reshape(B, S, D)

    r_y.wait_recv()
    r_x1 = remote_copy(oy, ox1, 2, x_nbr)
    r_x1.start()
    out_ref[...] += _dot(oy[...].reshape(B * S, G4), wo_rows(g ^ 1)).reshape(B, S, D)

    r_x0.wait_recv()
    out_ref[...] += _dot(ox0[...].reshape(B * S, G4), wo_rows(g ^ 2)).reshape(B, S, D)

    r_x1.wait_recv()
    out_ref[...] += _dot(ox1[...].reshape(B * S, G4), wo_rows(g ^ 3)).reshape(B, S, D)

    r_y.wait_send()
    r_x0.wait_send()
    r_x1.wait_send()


def _gather_project(O, Wo):
    return pl.pallas_call(
        _gather_body,
        out_shape=jax.ShapeDtypeStruct((B, S, D), jnp.float32),
        in_specs=[pl.BlockSpec(memory_space=pltpu.VMEM)] * 2,
        out_specs=pl.BlockSpec(memory_space=pltpu.VMEM),
        scratch_shapes=[
            pltpu.VMEM((B, S, G4), jnp.float32),
            pltpu.VMEM((B, S, G4), jnp.float32),
            pltpu.VMEM((B, S, G4), jnp.float32),
            pltpu.SemaphoreType.DMA((3,)),
            pltpu.SemaphoreType.DMA((3,)),
        ],
        compiler_params=pltpu.CompilerParams(
            collective_id=1, has_side_effects=True
        ),
    )(O, Wo)


def kernel(x, Wdkv, Wuk, Wuv, Wq, Wqr, Wkr, Wo):
    K, V = _compute_kv(x, Wdkv, Wuk, Wuv)
    g = (2 * lax.axis_index("x") + lax.axis_index("y")).astype(jnp.int32)
    O = _attention(jnp.reshape(g, (1,)), x, K, V, Wq, Wqr, Wkr)
    return _gather_project(O, Wo)


# device time: 105866 ns/iter; 1.6096x vs baseline; 1.3339x over previous
import jax
import jax.numpy as jnp
from jax import lax
from jax.experimental import pallas as pl
from jax.experimental.pallas import tpu as pltpu

B, S, D = 2, 512, 2048
H, Dh, Dr = 16, 128, 32
NJ = H // 4
G4 = NJ * Dh
DC_HALF = 128
SCALE = (Dh + Dr) ** -0.5


def _dot(a, b):
    return jnp.dot(a, b, preferred_element_type=jnp.float32)


def _dot_t(a, b):
    return lax.dot_general(
        a, b, (((1,), (1,)), ((), ())), preferred_element_type=jnp.float32
    )


def _kv_body(x_ref, wdkv_ref, wuk_ref, wuv_ref, wqr_ref,
             k_ref, v_ref, wqrt_ref,
             c_loc, c_rem, wuk_rem, wuv_rem, wuk_send, wuv_send,
             send_sems, recv_sems):
    my_x = lax.axis_index("x")
    my_y = lax.axis_index("y")
    g = 2 * my_x + my_y
    gn = 2 * my_x + (1 - my_y)
    nbr = (my_x, 1 - my_y)

    barrier = pltpu.get_barrier_semaphore()
    pl.semaphore_signal(
        barrier, inc=1, device_id=nbr, device_id_type=pl.DeviceIdType.MESH
    )
    pl.semaphore_wait(barrier, 1)

    xm = x_ref[...].reshape(B * S, D)
    c_loc[...] = _dot(xm, wdkv_ref[...])
    wuk_send[...] = wuk_ref[:, pl.ds(gn * G4, G4)]
    wuv_send[...] = wuv_ref[:, pl.ds(gn * G4, G4)]

    rdmas = []
    for src, dst, i in (
        (c_loc, c_rem, 0),
        (wuk_send, wuk_rem, 1),
        (wuv_send, wuv_rem, 2),
    ):
        rdma = pltpu.make_async_remote_copy(
            src_ref=src,
            dst_ref=dst,
            send_sem=send_sems.at[i],
            recv_sem=recv_sems.at[i],
            device_id=nbr,
            device_id_type=pl.DeviceIdType.MESH,
        )
        rdma.start()
        rdmas.append(rdma)

    wqrt_ref[...] = jnp.transpose(
        wqr_ref[:, pl.ds(g * NJ * Dr, NJ * Dr)]
    ).reshape(NJ, Dr, D)
    k_part = _dot(c_loc[...], wuk_ref[:, pl.ds(g * G4, G4)])
    v_part = _dot(c_loc[...], wuv_ref[:, pl.ds(g * G4, G4)])

    for rdma in rdmas:
        rdma.wait()

    k = k_part + _dot(c_rem[...], wuk_rem[...])
    v = v_part + _dot(c_rem[...], wuv_rem[...])
    k_ref[...] = k.reshape(B, S, G4)
    v_ref[...] = v.reshape(B, S, G4)


def _compute_kv(x, Wdkv, Wuk, Wuv, Wqr):
    return pl.pallas_call(
        _kv_body,
        out_shape=(
            jax.ShapeDtypeStruct((B, S, G4), jnp.float32),
            jax.ShapeDtypeStruct((B, S, G4), jnp.float32),
            jax.ShapeDtypeStruct((NJ, Dr, D), jnp.float32),
        ),
        in_specs=[pl.BlockSpec(memory_space=pltpu.VMEM)] * 5,
        out_specs=(
            pl.BlockSpec(memory_space=pltpu.VMEM),
            pl.BlockSpec(memory_space=pltpu.VMEM),
            pl.BlockSpec(memory_space=pltpu.VMEM),
        ),
        scratch_shapes=[
            pltpu.VMEM((B * S, DC_HALF), jnp.float32),
            pltpu.VMEM((B * S, DC_HALF), jnp.float32),
            pltpu.VMEM((DC_HALF, G4), jnp.float32),
            pltpu.VMEM((DC_HALF, G4), jnp.float32),
            pltpu.VMEM((DC_HALF, G4), jnp.float32),
            pltpu.VMEM((DC_HALF, G4), jnp.float32),
            pltpu.SemaphoreType.DMA((3,)),
            pltpu.SemaphoreType.DMA((3,)),
        ],
        compiler_params=pltpu.CompilerParams(
            collective_id=0, has_side_effects=True
        ),
    )(x, Wdkv, Wuk, Wuv, Wqr)


def _attn_body(g_ref, x_ref, k_ref, v_ref, wq_ref, wqrt_ref, wkr_ref,
               o_ref, kr_scratch):
    del g_ref
    j = pl.program_id(1)
    xb = x_ref[0]

    @pl.when(j == 0)
    def _():
        kr_scratch[...] = _dot(xb, wkr_ref[...])

    q = _dot(xb, wq_ref[...])
    qr = _dot_t(xb, wqrt_ref[0])

    scores = (_dot_t(q, k_ref[0]) + _dot_t(qr, kr_scratch[...])) * SCALE
    m = jnp.max(scores, axis=-1, keepdims=True)
    p = jnp.exp(scores - m)
    p = p / jnp.sum(p, axis=-1, keepdims=True)
    o_ref[0] = _dot(p, v_ref[0])


def _attention(g, x, K, V, Wq, Wqr_t, Wkr):
    grid_spec = pltpu.PrefetchScalarGridSpec(
        num_scalar_prefetch=1,
        grid=(B, NJ),
        in_specs=[
            pl.BlockSpec((1, S, D), lambda b, j, g: (b, 0, 0)),
            pl.BlockSpec((1, S, Dh), lambda b, j, g: (b, 0, j)),
            pl.BlockSpec((1, S, Dh), lambda b, j, g: (b, 0, j)),
            pl.BlockSpec((D, Dh), lambda b, j, g: (0, g[0] * NJ + j)),
            pl.BlockSpec((1, Dr, D), lambda b, j, g: (j, 0, 0)),
            pl.BlockSpec((D, Dr), lambda b, j, g: (0, 0)),
        ],
        out_specs=pl.BlockSpec((1, S, Dh), lambda b, j, g: (b, 0, j)),
        scratch_shapes=[pltpu.VMEM((S, Dr), jnp.float32)],
    )
    return pl.pallas_call(
        _attn_body,
        grid_spec=grid_spec,
        out_shape=jax.ShapeDtypeStruct((B, S, G4), jnp.float32),
        compiler_params=pltpu.CompilerParams(
            dimension_semantics=("arbitrary", "arbitrary"),
        ),
    )(g, x, K, V, Wq, Wqr_t, Wkr)


def _gather_body(o_ref, wo_ref, out_ref, oy, ox0, o3, send_sems, recv_sems):
    my_x = lax.axis_index("x")
    my_y = lax.axis_index("y")
    g = 2 * my_x + my_y
    y_nbr = (my_x, 1 - my_y)
    x_nbr = (1 - my_x, my_y)

    barrier = pltpu.get_barrier_semaphore()
    for nbr in (y_nbr, x_nbr):
        pl.semaphore_signal(
            barrier, inc=1, device_id=nbr, device_id_type=pl.DeviceIdType.MESH
        )
    pl.semaphore_wait(barrier, 2)

    def remote_copy(src, dst, i, dev):
        return pltpu.make_async_remote_copy(
            src_ref=src, dst_ref=dst,
            send_sem=send_sems.at[i], recv_sem=recv_sems.at[i],
            device_id=dev, device_id_type=pl.DeviceIdType.MESH,
        )

    r_y = remote_copy(o_ref, oy, 0, y_nbr)
    r_x0 = remote_copy(o_ref, ox0, 1, x_nbr)
    r_y.start()
    r_x0.start()

    def wo_rows(grp):
        return wo_ref[pl.ds(grp * G4, G4), :]

    out_ref[...] = _dot(o_ref[...].reshape(B * S, G4), wo_rows(g)).reshape(B, S, D)

    r_y.wait_recv()
    r3 = remote_copy(oy.at[0], o3.at[0], 2, x_nbr)
    r3.start()
    out_ref[...] += _dot(oy[...].reshape(B * S, G4), wo_rows(g ^ 1)).reshape(B, S, D)

    r_x0.wait_recv()
    r4 = remote_copy(ox0.at[1], o3.at[1], 3, y_nbr)
    r4.start()
    out_ref[...] += _dot(ox0[...].reshape(B * S, G4), wo_rows(g ^ 2)).reshape(B, S, D)

    r3.wait_recv()
    out_ref[0] += _dot(o3[0], wo_rows(g ^ 3))
    r4.wait_recv()
    out_ref[1] += _dot(o3[1], wo_rows(g ^ 3))

    for r in (r_y, r_x0, r3, r4):
        r.wait_send()


def _gather_project(O, Wo):
    return pl.pallas_call(
        _gather_body,
        out_shape=jax.ShapeDtypeStruct((B, S, D), jnp.float32),
        in_specs=[pl.BlockSpec(memory_space=pltpu.VMEM)] * 2,
        out_specs=pl.BlockSpec(memory_space=pltpu.VMEM),
        scratch_shapes=[
            pltpu.VMEM((B, S, G4), jnp.float32),
            pltpu.VMEM((B, S, G4), jnp.float32),
            pltpu.VMEM((B, S, G4), jnp.float32),
            pltpu.SemaphoreType.DMA((4,)),
            pltpu.SemaphoreType.DMA((4,)),
        ],
        compiler_params=pltpu.CompilerParams(
            collective_id=1, has_side_effects=True
        ),
    )(O, Wo)


def kernel(x, Wdkv, Wuk, Wuv, Wq, Wqr, Wkr, Wo):
    K, V, Wqr_t = _compute_kv(x, Wdkv, Wuk, Wuv, Wqr)
    g = (2 * lax.axis_index("x") + lax.axis_index("y")).astype(jnp.int32)
    O = _attention(jnp.reshape(g, (1,)), x, K, V, Wq, Wqr_t, Wkr)
    return _gather_project(O, Wo)
